# baseline (speedup 1.0000x reference)
import math

import jax
import jax.numpy as jnp
from jax import lax
from jax.experimental import pallas as pl
from jax.experimental.pallas import tpu as pltpu
from jax.experimental.pallas import tpu_sc as plsc

_CUTOFF = 5.0
_LOG2 = math.log(2.0)

_NC, _NS = 2, 16
_NW = _NC * _NS


_LOG2E = 1.4426950408889634


def _ssp(v):
    t = jnp.exp2(jnp.abs(v) * (-_LOG2E))
    return jnp.maximum(v, 0.0) + jnp.log2(1.0 + t) * _LOG2 - _LOG2


def _in2f_body(x_ref, nbh_ref, win_ref, y_ref, nflat_ref):
    b = pl.program_id(0)
    na = x_ref.shape[1]
    y_ref[...] = jnp.dot(x_ref[0], win_ref[...],
                         preferred_element_type=jnp.float32)
    nflat_ref[0] = nbh_ref[0] + b * na


def _in2f(x, neighbors, win):
    nb, na, nab = x.shape
    nnbh = neighbors.shape[2]
    nf = win.shape[1]
    return pl.pallas_call(
        _in2f_body,
        grid=(nb,),
        in_specs=[
            pl.BlockSpec((1, na, nab), lambda b: (b, 0, 0)),
            pl.BlockSpec((1, na, nnbh), lambda b: (b, 0, 0)),
            pl.BlockSpec((nab, nf), lambda b: (0, 0)),
        ],
        out_specs=[
            pl.BlockSpec((na, nf), lambda b: (b, 0)),
            pl.BlockSpec((1, na, nnbh), lambda b: (b, 0, 0)),
        ],
        out_shape=[
            jax.ShapeDtypeStruct((nb * na, nf), jnp.float32),
            jax.ShapeDtypeStruct((nb, na, nnbh), jnp.int32),
        ],
    )(x, neighbors, win)


def _sc_gather(table, idx3):
    nw, nch, ch = idx3.shape
    nf = table.shape[1]
    per_w = nch * ch
    mesh = plsc.VectorSubcoreMesh(core_axis_name="c", subcore_axis_name="s")

    def body(table_ref, idx_ref, out_ref, idx_v, rows_v, sem0, sem1):
        wid = lax.axis_index("s") * _NC + lax.axis_index("c")
        pltpu.sync_copy(idx_ref.at[wid], idx_v)
        base = wid * per_w
        sems = (sem0, sem1)
        pltpu.async_copy(table_ref.at[idx_v.at[0]], rows_v.at[0], sem0)

        def step(c2, carry):
            c0 = c2 * 2
            for k in (0, 1):
                c = c0 + k
                pltpu.make_async_copy(
                    table_ref.at[idx_v.at[c]], rows_v.at[k], sems[k]
                ).wait()

                @pl.when(c + 1 < nch)
                def _():
                    pltpu.async_copy(
                        table_ref.at[idx_v.at[c + 1]], rows_v.at[1 - k], sems[1 - k]
                    )

                pltpu.sync_copy(
                    rows_v.at[k], out_ref.at[pl.ds(base + c * ch, ch)]
                )
            return carry

        lax.fori_loop(0, nch // 2, step, 0)
        if nch % 2:
            c = nch - 1
            pltpu.make_async_copy(
                table_ref.at[idx_v.at[c]], rows_v.at[0], sem0
            ).wait()
            pltpu.sync_copy(rows_v.at[0], out_ref.at[pl.ds(base + c * ch, ch)])

    return pl.kernel(
        body,
        out_type=jax.ShapeDtypeStruct((nw * per_w, nf), table.dtype),
        mesh=mesh,
        scratch_types=[
            pltpu.VMEM((nch, ch), jnp.int32),
            pltpu.VMEM((2, ch, nf), table.dtype),
            pltpu.SemaphoreType.DMA,
            pltpu.SemaphoreType.DMA,
        ],
    )(table, idx3)


def _fused_body(g_ref, f_ref, r_ref, m_ref, w1_ref, b1_ref, w2_ref, b2_ref,
                wf2_ref, bf2_ref, wd_ref, bd_ref, out_ref):
    _, ba, nnbh, ns = f_ref.shape
    nf = w2_ref.shape[-1]
    f2 = f_ref[...].reshape(ba * nnbh, ns)
    h = jnp.dot(f2, w1_ref[...], preferred_element_type=jnp.float32)
    h = _ssp(h + b1_ref[...])
    w = jnp.dot(h, w2_ref[...], preferred_element_type=jnp.float32) + b2_ref[...]
    r = r_ref[0]
    cm = 0.5 * (jnp.cos(r * (math.pi / _CUTOFF)) + 1.0)
    cm = cm * (r < _CUTOFF).astype(jnp.float32) * m_ref[0]
    w3 = w.reshape(ba, nnbh, nf) * cm[:, :, None]
    g3 = g_ref[...].reshape(ba, nnbh, nf)
    s = jnp.sum(w3 * g3, axis=1)
    z = _ssp(jnp.dot(s, wf2_ref[...], preferred_element_type=jnp.float32)
             + bf2_ref[...])
    out_ref[0] = (jnp.dot(z, wd_ref[...], preferred_element_type=jnp.float32)
                  + bd_ref[...])


def _fused(g, f_ij, r_ij, mask, w1, b1, w2, b2, wf2, bf2, wd, bd, ba, b0, nbs):
    nb, na, nnbh, ns = f_ij.shape
    nf = w2.shape[-1]
    nj = na // ba
    full = lambda b, j: (0, 0)
    return pl.pallas_call(
        _fused_body,
        grid=(nbs, nj),
        in_specs=[
            pl.BlockSpec((ba * nnbh, nf), lambda b, j: (b * nj + j, 0)),
            pl.BlockSpec((1, ba, nnbh, ns), lambda b, j: (b + b0, j, 0, 0)),
            pl.BlockSpec((1, ba, nnbh), lambda b, j: (b + b0, j, 0)),
            pl.BlockSpec((1, ba, nnbh), lambda b, j: (b + b0, j, 0)),
            pl.BlockSpec(w1.shape, full),
            pl.BlockSpec(b1.shape, full),
            pl.BlockSpec(w2.shape, full),
            pl.BlockSpec(b2.shape, full),
            pl.BlockSpec(wf2.shape, full),
            pl.BlockSpec(bf2.shape, full),
            pl.BlockSpec(wd.shape, full),
            pl.BlockSpec(bd.shape, full),
        ],
        out_specs=pl.BlockSpec((1, ba, nf), lambda b, j: (b, j, 0)),
        out_shape=jax.ShapeDtypeStruct((nbs, na, nf), jnp.float32),
    )(g, f_ij, r_ij, mask, w1, b1, w2, b2, wf2, bf2, wd, bd)


def kernel(x, r_ij, neighbors, neighbor_mask, f_ij, W1, b1, W2, b2, Win, Wf2,
           bf2, Wd, bd):
    nb, na, nnbh = neighbors.shape
    nf = Win.shape[1]
    nrow = nb * na
    ne = nrow * nnbh

    y2, nflat = _in2f(x, neighbors, Win)

    nslice = 2
    nbs = nb // nslice
    per_w = nbs * na * nnbh // _NW
    ch = 40
    nch = per_w // ch
    ba = 200
    outs = []
    for s in range(nslice):
        idx3 = nflat[s * nbs:(s + 1) * nbs].reshape(_NW, nch, ch)
        g = _sc_gather(y2, idx3)
        outs.append(_fused(
            g, f_ij, r_ij, neighbor_mask,
            W1, b1.reshape(1, nf), W2, b2.reshape(1, nf),
            Wf2, bf2.reshape(1, nf), Wd, bd.reshape(1, nf),
            ba, s * nbs, nbs,
        ))
    return jnp.concatenate(outs, axis=0)


# speedup vs baseline: 1.1215x; 1.1215x over previous
import math

import jax
import jax.numpy as jnp
from jax import lax
from jax.experimental import pallas as pl
from jax.experimental.pallas import tpu as pltpu
from jax.experimental.pallas import tpu_sc as plsc

_CUTOFF = 5.0
_LOG2 = math.log(2.0)

_NC, _NS = 2, 16
_NW = _NC * _NS


_LOG2E = 1.4426950408889634


def _ssp(v):
    t = jnp.exp2(jnp.abs(v) * (-_LOG2E))
    return jnp.maximum(v, 0.0) + jnp.log2(1.0 + t) * _LOG2 - _LOG2


def _in2f_body(x_ref, nbh_ref, win_ref, y_ref, nflat_ref):
    b = pl.program_id(0)
    na = x_ref.shape[1]
    y_ref[...] = jnp.dot(x_ref[0], win_ref[...],
                         preferred_element_type=jnp.float32)
    nflat_ref[0] = nbh_ref[0] + b * na


def _in2f(x, neighbors, win):
    nb, na, nab = x.shape
    nnbh = neighbors.shape[2]
    nf = win.shape[1]
    return pl.pallas_call(
        _in2f_body,
        grid=(nb,),
        in_specs=[
            pl.BlockSpec((1, na, nab), lambda b: (b, 0, 0)),
            pl.BlockSpec((1, na, nnbh), lambda b: (b, 0, 0)),
            pl.BlockSpec((nab, nf), lambda b: (0, 0)),
        ],
        out_specs=[
            pl.BlockSpec((na, nf), lambda b: (b, 0)),
            pl.BlockSpec((1, na, nnbh), lambda b: (b, 0, 0)),
        ],
        out_shape=[
            jax.ShapeDtypeStruct((nb * na, nf), jnp.float32),
            jax.ShapeDtypeStruct((nb, na, nnbh), jnp.int32),
        ],
    )(x, neighbors, win)


def _sc_gather(table, idx3):
    nw, nch, ch = idx3.shape
    nf = table.shape[1]
    per_w = nch * ch
    mesh = plsc.VectorSubcoreMesh(core_axis_name="c", subcore_axis_name="s")

    def body(table_ref, idx_ref, out_ref, idx_v, rows_v,
             gsem0, gsem1, osem0, osem1):
        wid = lax.axis_index("s") * _NC + lax.axis_index("c")
        pltpu.sync_copy(idx_ref.at[wid], idx_v)
        base = wid * per_w
        gsems = (gsem0, gsem1)
        osems = (osem0, osem1)

        def gather(c, k):
            pltpu.async_copy(table_ref.at[idx_v.at[c]], rows_v.at[k], gsems[k])

        def wait_gather(c, k):
            pltpu.make_async_copy(
                table_ref.at[idx_v.at[c]], rows_v.at[k], gsems[k]).wait()

        def put(c, k):
            pltpu.async_copy(
                rows_v.at[k], out_ref.at[pl.ds(base + c * ch, ch)], osems[k])

        def wait_put(c, k):
            pltpu.make_async_copy(
                rows_v.at[k], out_ref.at[pl.ds(base + c * ch, ch)],
                osems[k]).wait()

        gather(0, 0)

        def step(c2, carry):
            c0 = c2 * 2
            for k in (0, 1):
                c = c0 + k

                @pl.when(c < nch)
                def _():
                    wait_gather(c, k)

                    @pl.when(c + 1 < nch)
                    def _():
                        @pl.when(c > 0)
                        def _():
                            wait_put(c - 1, 1 - k)
                        gather(c + 1, 1 - k)

                    put(c, k)
            return carry

        lax.fori_loop(0, (nch + 1) // 2, step, 0)
        last = nch - 1
        wait_put(last, last % 2)
        if nch > 1:
            wait_put(last - 1, (last - 1) % 2)

    return pl.kernel(
        body,
        out_type=jax.ShapeDtypeStruct((nw * per_w, nf), table.dtype),
        mesh=mesh,
        scratch_types=[
            pltpu.VMEM((nch, ch), jnp.int32),
            pltpu.VMEM((2, ch, nf), table.dtype),
            pltpu.SemaphoreType.DMA,
            pltpu.SemaphoreType.DMA,
            pltpu.SemaphoreType.DMA,
            pltpu.SemaphoreType.DMA,
        ],
    )(table, idx3)


def _fused_body(g_ref, f_ref, r_ref, m_ref, w1_ref, b1_ref, w2_ref, b2_ref,
                wf2_ref, bf2_ref, wd_ref, bd_ref, out_ref):
    _, ba, nnbh, ns = f_ref.shape
    nf = w2_ref.shape[-1]
    f2 = f_ref[...].reshape(ba * nnbh, ns)
    h = jnp.dot(f2, w1_ref[...], preferred_element_type=jnp.float32)
    h = _ssp(h + b1_ref[...])
    w = jnp.dot(h, w2_ref[...], preferred_element_type=jnp.float32) + b2_ref[...]
    r = r_ref[0]
    cm = 0.5 * (jnp.cos(r * (math.pi / _CUTOFF)) + 1.0)
    cm = cm * (r < _CUTOFF).astype(jnp.float32) * m_ref[0]
    w3 = w.reshape(ba, nnbh, nf) * cm[:, :, None]
    g3 = g_ref[...].reshape(ba, nnbh, nf)
    s = jnp.sum(w3 * g3, axis=1)
    z = _ssp(jnp.dot(s, wf2_ref[...], preferred_element_type=jnp.float32)
             + bf2_ref[...])
    out_ref[0] = (jnp.dot(z, wd_ref[...], preferred_element_type=jnp.float32)
                  + bd_ref[...])


def _fused(g, f_ij, r_ij, mask, w1, b1, w2, b2, wf2, bf2, wd, bd, ba, b0, nbs):
    nb, na, nnbh, ns = f_ij.shape
    nf = w2.shape[-1]
    nj = na // ba
    full = lambda b, j: (0, 0)
    return pl.pallas_call(
        _fused_body,
        grid=(nbs, nj),
        in_specs=[
            pl.BlockSpec((ba * nnbh, nf), lambda b, j: (b * nj + j, 0)),
            pl.BlockSpec((1, ba, nnbh, ns), lambda b, j: (b + b0, j, 0, 0)),
            pl.BlockSpec((1, ba, nnbh), lambda b, j: (b + b0, j, 0)),
            pl.BlockSpec((1, ba, nnbh), lambda b, j: (b + b0, j, 0)),
            pl.BlockSpec(w1.shape, full),
            pl.BlockSpec(b1.shape, full),
            pl.BlockSpec(w2.shape, full),
            pl.BlockSpec(b2.shape, full),
            pl.BlockSpec(wf2.shape, full),
            pl.BlockSpec(bf2.shape, full),
            pl.BlockSpec(wd.shape, full),
            pl.BlockSpec(bd.shape, full),
        ],
        out_specs=pl.BlockSpec((1, ba, nf), lambda b, j: (b, j, 0)),
        out_shape=jax.ShapeDtypeStruct((nbs, na, nf), jnp.float32),
    )(g, f_ij, r_ij, mask, w1, b1, w2, b2, wf2, bf2, wd, bd)


def kernel(x, r_ij, neighbors, neighbor_mask, f_ij, W1, b1, W2, b2, Win, Wf2,
           bf2, Wd, bd):
    nb, na, nnbh = neighbors.shape
    nf = Win.shape[1]
    nrow = nb * na
    ne = nrow * nnbh

    y2, nflat = _in2f(x, neighbors, Win)

    per_w = ne // _NW
    ch = 80
    nch = per_w // ch
    idx3 = nflat.reshape(_NW, nch, ch)
    g = _sc_gather(y2, idx3)

    ba = 200
    return _fused(
        g, f_ij, r_ij, neighbor_mask,
        W1, b1.reshape(1, nf), W2, b2.reshape(1, nf),
        Wf2, bf2.reshape(1, nf), Wd, bd.reshape(1, nf),
        ba, 0, nb,
    )


# speedup vs baseline: 1.1844x; 1.0560x over previous
import math

import jax
import jax.numpy as jnp
from jax import lax
from jax.experimental import pallas as pl
from jax.experimental.pallas import tpu as pltpu
from jax.experimental.pallas import tpu_sc as plsc

_CUTOFF = 5.0
_LOG2 = math.log(2.0)

_NC, _NS = 2, 16
_NW = _NC * _NS


_LOG2E = 1.4426950408889634


def _ssp(v):
    t = jnp.exp2(jnp.abs(v) * (-_LOG2E))
    return jnp.maximum(v, 0.0) + jnp.log2(1.0 + t) * _LOG2 - _LOG2


def _in2f_body(x_ref, nbh_ref, win_ref, y_ref, nflat_ref):
    b = pl.program_id(0)
    na = x_ref.shape[1]
    y_ref[...] = jnp.dot(x_ref[0], win_ref[...],
                         preferred_element_type=jnp.float32)
    nflat_ref[0] = nbh_ref[0] + b * na


def _in2f(x, neighbors, win):
    nb, na, nab = x.shape
    nnbh = neighbors.shape[2]
    nf = win.shape[1]
    return pl.pallas_call(
        _in2f_body,
        grid=(nb,),
        in_specs=[
            pl.BlockSpec((1, na, nab), lambda b: (b, 0, 0)),
            pl.BlockSpec((1, na, nnbh), lambda b: (b, 0, 0)),
            pl.BlockSpec((nab, nf), lambda b: (0, 0)),
        ],
        out_specs=[
            pl.BlockSpec((na, nf), lambda b: (b, 0)),
            pl.BlockSpec((1, na, nnbh), lambda b: (b, 0, 0)),
        ],
        out_shape=[
            jax.ShapeDtypeStruct((nb * na, nf), jnp.float32),
            jax.ShapeDtypeStruct((nb, na, nnbh), jnp.int32),
        ],
    )(x, neighbors, win)


def _sc_gather(table, idx3):
    nw, nch, ch = idx3.shape
    nf = table.shape[1]
    per_w = nch * ch
    mesh = plsc.VectorSubcoreMesh(core_axis_name="c", subcore_axis_name="s")

    def body(table_ref, idx_ref, out_ref, idx_v, rows_v,
             gsem0, gsem1, osem0, osem1):
        wid = lax.axis_index("s") * _NC + lax.axis_index("c")
        pltpu.sync_copy(idx_ref.at[wid], idx_v)
        base = wid * per_w
        gsems = (gsem0, gsem1)
        osems = (osem0, osem1)

        def gather(c, k):
            pltpu.async_copy(table_ref.at[idx_v.at[c]], rows_v.at[k], gsems[k])

        def wait_gather(c, k):
            pltpu.make_async_copy(
                table_ref.at[idx_v.at[c]], rows_v.at[k], gsems[k]).wait()

        def put(c, k):
            pltpu.async_copy(
                rows_v.at[k], out_ref.at[pl.ds(base + c * ch, ch)], osems[k])

        def wait_put(c, k):
            pltpu.make_async_copy(
                rows_v.at[k], out_ref.at[pl.ds(base + c * ch, ch)],
                osems[k]).wait()

        gather(0, 0)

        def step(c2, carry):
            c0 = c2 * 2
            for k in (0, 1):
                c = c0 + k

                @pl.when(c < nch)
                def _():
                    @pl.when(c + 1 < nch)
                    def _():
                        @pl.when(c > 0)
                        def _():
                            wait_put(c - 1, 1 - k)
                        gather(c + 1, 1 - k)

                    wait_gather(c, k)
                    put(c, k)
            return carry

        lax.fori_loop(0, (nch + 1) // 2, step, 0)
        last = nch - 1
        wait_put(last, last % 2)
        if nch > 1:
            wait_put(last - 1, (last - 1) % 2)

    return pl.kernel(
        body,
        out_type=jax.ShapeDtypeStruct((nw * per_w, nf), table.dtype),
        mesh=mesh,
        scratch_types=[
            pltpu.VMEM((nch, ch), jnp.int32),
            pltpu.VMEM((2, ch, nf), table.dtype),
            pltpu.SemaphoreType.DMA,
            pltpu.SemaphoreType.DMA,
            pltpu.SemaphoreType.DMA,
            pltpu.SemaphoreType.DMA,
        ],
    )(table, idx3)


def _fused_body(g_ref, f_ref, r_ref, m_ref, w1_ref, b1_ref, w2_ref, b2_ref,
                wf2_ref, bf2_ref, wd_ref, bd_ref, out_ref):
    _, ba, nnbh, ns = f_ref.shape
    nf = w2_ref.shape[-1]
    f2 = f_ref[...].reshape(ba * nnbh, ns)
    h = jnp.dot(f2, w1_ref[...], preferred_element_type=jnp.float32)
    h = _ssp(h + b1_ref[...])
    w = jnp.dot(h, w2_ref[...], preferred_element_type=jnp.float32) + b2_ref[...]
    r = r_ref[0]
    cm = 0.5 * (jnp.cos(r * (math.pi / _CUTOFF)) + 1.0)
    cm = cm * (r < _CUTOFF).astype(jnp.float32) * m_ref[0]
    w3 = w.reshape(ba, nnbh, nf) * cm[:, :, None]
    g3 = g_ref[...].reshape(ba, nnbh, nf)
    s = jnp.sum(w3 * g3, axis=1)
    z = _ssp(jnp.dot(s, wf2_ref[...], preferred_element_type=jnp.float32)
             + bf2_ref[...])
    out_ref[0] = (jnp.dot(z, wd_ref[...], preferred_element_type=jnp.float32)
                  + bd_ref[...])


def _fused(g, f_ij, r_ij, mask, w1, b1, w2, b2, wf2, bf2, wd, bd, ba, b0, nbs):
    nb, na, nnbh, ns = f_ij.shape
    nf = w2.shape[-1]
    nj = na // ba
    full = lambda b, j: (0, 0)
    return pl.pallas_call(
        _fused_body,
        grid=(nbs, nj),
        in_specs=[
            pl.BlockSpec((ba * nnbh, nf), lambda b, j: (b * nj + j, 0)),
            pl.BlockSpec((1, ba, nnbh, ns), lambda b, j: (b + b0, j, 0, 0)),
            pl.BlockSpec((1, ba, nnbh), lambda b, j: (b + b0, j, 0)),
            pl.BlockSpec((1, ba, nnbh), lambda b, j: (b + b0, j, 0)),
            pl.BlockSpec(w1.shape, full),
            pl.BlockSpec(b1.shape, full),
            pl.BlockSpec(w2.shape, full),
            pl.BlockSpec(b2.shape, full),
            pl.BlockSpec(wf2.shape, full),
            pl.BlockSpec(bf2.shape, full),
            pl.BlockSpec(wd.shape, full),
            pl.BlockSpec(bd.shape, full),
        ],
        out_specs=pl.BlockSpec((1, ba, nf), lambda b, j: (b, j, 0)),
        out_shape=jax.ShapeDtypeStruct((nbs, na, nf), jnp.float32),
    )(g, f_ij, r_ij, mask, w1, b1, w2, b2, wf2, bf2, wd, bd)


def kernel(x, r_ij, neighbors, neighbor_mask, f_ij, W1, b1, W2, b2, Win, Wf2,
           bf2, Wd, bd):
    nb, na, nnbh = neighbors.shape
    nf = Win.shape[1]
    nrow = nb * na
    ne = nrow * nnbh

    y2, nflat = _in2f(x, neighbors, Win)

    per_w = ne // _NW
    ch = 80
    nch = per_w // ch
    idx3 = nflat.reshape(_NW, nch, ch)
    g = _sc_gather(y2, idx3)

    ba = 200
    return _fused(
        g, f_ij, r_ij, neighbor_mask,
        W1, b1.reshape(1, nf), W2, b2.reshape(1, nf),
        Wf2, bf2.reshape(1, nf), Wd, bd.reshape(1, nf),
        ba, 0, nb,
    )


# speedup vs baseline: 1.1847x; 1.0003x over previous
import math

import jax
import jax.numpy as jnp
from jax import lax
from jax.experimental import pallas as pl
from jax.experimental.pallas import tpu as pltpu
from jax.experimental.pallas import tpu_sc as plsc

_CUTOFF = 5.0
_LOG2 = math.log(2.0)

_NC, _NS = 2, 16
_NW = _NC * _NS


_LOG2E = 1.4426950408889634


def _ssp(v):
    t = jnp.exp2(jnp.abs(v) * (-_LOG2E))
    return jnp.maximum(v, 0.0) + jnp.log2(1.0 + t) * _LOG2 - _LOG2


def _in2f_body(x_ref, nbh_ref, win_ref, y_ref, nflat_ref):
    b = pl.program_id(0)
    na = x_ref.shape[1]
    y_ref[...] = jnp.dot(x_ref[0], win_ref[...],
                         preferred_element_type=jnp.float32)
    nflat_ref[0] = nbh_ref[0] + b * na


def _in2f(x, neighbors, win):
    nb, na, nab = x.shape
    nnbh = neighbors.shape[2]
    nf = win.shape[1]
    return pl.pallas_call(
        _in2f_body,
        grid=(nb,),
        in_specs=[
            pl.BlockSpec((1, na, nab), lambda b: (b, 0, 0)),
            pl.BlockSpec((1, na, nnbh), lambda b: (b, 0, 0)),
            pl.BlockSpec((nab, nf), lambda b: (0, 0)),
        ],
        out_specs=[
            pl.BlockSpec((na, nf), lambda b: (b, 0)),
            pl.BlockSpec((1, na, nnbh), lambda b: (b, 0, 0)),
        ],
        out_shape=[
            jax.ShapeDtypeStruct((nb * na, nf), jnp.float32),
            jax.ShapeDtypeStruct((nb, na, nnbh), jnp.int32),
        ],
    )(x, neighbors, win)


def _sc_gather(table, idx3):
    nw, nch, ch = idx3.shape
    nf = table.shape[1]
    per_w = nch * ch
    mesh = plsc.VectorSubcoreMesh(core_axis_name="c", subcore_axis_name="s")

    nbuf = 4

    def body(table_ref, idx_ref, out_ref, idx_v, rows_v, *sems):
        wid = lax.axis_index("s") * _NC + lax.axis_index("c")
        pltpu.sync_copy(idx_ref.at[wid], idx_v)
        base = wid * per_w
        gsems = sems[:nbuf]
        osems = sems[nbuf:]

        def gather(c, k):
            pltpu.async_copy(table_ref.at[idx_v.at[c]], rows_v.at[k], gsems[k])

        def wait_gather(c, k):
            pltpu.make_async_copy(
                table_ref.at[idx_v.at[c]], rows_v.at[k], gsems[k]).wait()

        def put(c, k):
            pltpu.async_copy(
                rows_v.at[k], out_ref.at[pl.ds(base + c * ch, ch)], osems[k])

        def wait_put(c, k):
            pltpu.make_async_copy(
                rows_v.at[k], out_ref.at[pl.ds(base + c * ch, ch)],
                osems[k]).wait()

        gather(0, 0)
        if nch > 1:
            gather(1, 1)

        def step(c4, carry):
            c0 = c4 * nbuf
            for k in range(nbuf):
                c = c0 + k

                @pl.when(c < nch)
                def _():
                    k2 = (k + 2) % nbuf

                    @pl.when(c + 2 < nch)
                    def _():
                        @pl.when(c > 1)
                        def _():
                            wait_put(c - 2, k2)
                        gather(c + 2, k2)

                    wait_gather(c, k)
                    put(c, k)
            return carry

        lax.fori_loop(0, (nch + nbuf - 1) // nbuf, step, 0)
        for c in range(max(0, nch - 4), nch):
            wait_put(c, c % nbuf)

    return pl.kernel(
        body,
        out_type=jax.ShapeDtypeStruct((nw * per_w, nf), table.dtype),
        mesh=mesh,
        scratch_types=[
            pltpu.VMEM((nch, ch), jnp.int32),
            pltpu.VMEM((nbuf, ch, nf), table.dtype),
        ] + [pltpu.SemaphoreType.DMA] * (2 * nbuf),
    )(table, idx3)


def _fused_body(g_ref, f_ref, r_ref, m_ref, w1_ref, b1_ref, w2_ref, b2_ref,
                wf2_ref, bf2_ref, wd_ref, bd_ref, out_ref):
    _, ba, nnbh, ns = f_ref.shape
    nf = w2_ref.shape[-1]
    f2 = f_ref[...].reshape(ba * nnbh, ns)
    h = jnp.dot(f2, w1_ref[...], preferred_element_type=jnp.float32)
    h = _ssp(h + b1_ref[...])
    w = jnp.dot(h, w2_ref[...], preferred_element_type=jnp.float32) + b2_ref[...]
    r = r_ref[0]
    cm = 0.5 * (jnp.cos(r * (math.pi / _CUTOFF)) + 1.0)
    cm = cm * (r < _CUTOFF).astype(jnp.float32) * m_ref[0]
    w3 = w.reshape(ba, nnbh, nf) * cm[:, :, None]
    g3 = g_ref[...].reshape(ba, nnbh, nf)
    s = jnp.sum(w3 * g3, axis=1)
    z = _ssp(jnp.dot(s, wf2_ref[...], preferred_element_type=jnp.float32)
             + bf2_ref[...])
    out_ref[0] = (jnp.dot(z, wd_ref[...], preferred_element_type=jnp.float32)
                  + bd_ref[...])


def _fused(g, f_ij, r_ij, mask, w1, b1, w2, b2, wf2, bf2, wd, bd, ba, b0, nbs):
    nb, na, nnbh, ns = f_ij.shape
    nf = w2.shape[-1]
    nj = na // ba
    full = lambda b, j: (0, 0)
    return pl.pallas_call(
        _fused_body,
        grid=(nbs, nj),
        in_specs=[
            pl.BlockSpec((ba * nnbh, nf), lambda b, j: (b * nj + j, 0)),
            pl.BlockSpec((1, ba, nnbh, ns), lambda b, j: (b + b0, j, 0, 0)),
            pl.BlockSpec((1, ba, nnbh), lambda b, j: (b + b0, j, 0)),
            pl.BlockSpec((1, ba, nnbh), lambda b, j: (b + b0, j, 0)),
            pl.BlockSpec(w1.shape, full),
            pl.BlockSpec(b1.shape, full),
            pl.BlockSpec(w2.shape, full),
            pl.BlockSpec(b2.shape, full),
            pl.BlockSpec(wf2.shape, full),
            pl.BlockSpec(bf2.shape, full),
            pl.BlockSpec(wd.shape, full),
            pl.BlockSpec(bd.shape, full),
        ],
        out_specs=pl.BlockSpec((1, ba, nf), lambda b, j: (b, j, 0)),
        out_shape=jax.ShapeDtypeStruct((nbs, na, nf), jnp.float32),
    )(g, f_ij, r_ij, mask, w1, b1, w2, b2, wf2, bf2, wd, bd)


def kernel(x, r_ij, neighbors, neighbor_mask, f_ij, W1, b1, W2, b2, Win, Wf2,
           bf2, Wd, bd):
    nb, na, nnbh = neighbors.shape
    nf = Win.shape[1]
    nrow = nb * na
    ne = nrow * nnbh

    y2, nflat = _in2f(x, neighbors, Win)

    per_w = ne // _NW
    ch = 80
    nch = per_w // ch
    idx3 = nflat.reshape(_NW, nch, ch)
    g = _sc_gather(y2, idx3)

    ba = 200
    return _fused(
        g, f_ij, r_ij, neighbor_mask,
        W1, b1.reshape(1, nf), W2, b2.reshape(1, nf),
        Wf2, bf2.reshape(1, nf), Wd, bd.reshape(1, nf),
        ba, 0, nb,
    )
